# baseline (speedup 1.0000x reference)
import functools

import jax
import jax.numpy as jnp
from jax import lax
from jax.experimental import pallas as pl
from jax.experimental.pallas import tpu as pltpu
from jax.experimental.pallas import tpu_sc as plsc

_DELTA_V = 0.5
_DELTA_D = 1.5
_NL = 20
_NS = 8
_SEG = _NL * _NS

_SC_WORKERS = 32
_LANES = 16
_HBINS = 192


def _sc_histogram(lab_pad, sb_pad, npad):
    chunk = npad // _SC_WORKERS
    rows = chunk // 128
    mesh = plsc.VectorSubcoreMesh(core_axis_name="c", subcore_axis_name="s")

    @functools.partial(
        pl.kernel, mesh=mesh,
        out_type=jax.ShapeDtypeStruct((2, _HBINS), jnp.float32),
        scratch_types=[
            pltpu.VMEM((chunk,), jnp.int32),
            pltpu.VMEM((chunk,), jnp.int32),
            pltpu.VMEM((rows, 128), jnp.int32),
            pltpu.VMEM((128,), jnp.float32),
            pltpu.VMEM((_HBINS,), jnp.float32),
            pltpu.VMEM_SHARED((_HBINS,), jnp.float32),
        ],
    )
    def hist(lab_hbm, sb_hbm, out_hbm,
             lab_v, sb_v, segs_v, ones_v, bins_v, shared_v):
        cid = lax.axis_index("c")
        sid = lax.axis_index("s")
        wid = sid * 2 + cid
        base = wid * chunk
        pltpu.sync_copy(lab_hbm.at[pl.ds(base, chunk)], lab_v)
        pltpu.sync_copy(sb_hbm.at[pl.ds(base, chunk)], sb_v)

        zeros16 = jnp.zeros((_LANES,), jnp.float32)
        ones16 = jnp.ones((_LANES,), jnp.float32)
        for t in range(128 // _LANES):
            ones_v[pl.ds(t * _LANES, _LANES)] = ones16
        for t in range(_HBINS // _LANES):
            bins_v[pl.ds(t * _LANES, _LANES)] = zeros16

        for t in range(chunk // _LANES):
            lab16 = lab_v[pl.ds(t * _LANES, _LANES)]
            sb16 = sb_v[pl.ds(t * _LANES, _LANES)]
            segs_v[t // 8, pl.ds((t % 8) * _LANES, _LANES)] = sb16 * _NL + lab16

        @pl.when(sid == 0)
        def _zero_shared():
            pltpu.sync_copy(bins_v, shared_v)
        plsc.subcore_barrier()

        for j in range(rows):
            pltpu.sync_copy(ones_v, shared_v.at[segs_v.at[j]], add=True)
        plsc.subcore_barrier()

        @pl.when(sid == 0)
        def _write_out():
            pltpu.sync_copy(shared_v, bins_v)
            pltpu.sync_copy(bins_v, out_hbm.at[cid])

    return hist(lab_pad, sb_pad)


def _dot(a, b, dims):
    return lax.dot_general(a, b, (dims, ((), ())),
                           preferred_element_type=jnp.float32)


def _loss_body(lab_ref, sb_ref, x_ref, cnt32_ref, out_ref,
               sums_ref, pull_ref, coef_ref, meta_ref, xn_ref,
               *, nb, r):
    phase = pl.program_id(0)
    i = pl.program_id(1)

    @pl.when((phase == 0) & (i == 0))
    def _init():
        sums_ref[...] = jnp.zeros_like(sums_ref)
        pull_ref[...] = jnp.zeros_like(pull_ref)

    lab = lab_ref[0]
    sb = sb_ref[0]
    seg = sb * _NL + lab
    seg_b = jnp.broadcast_to(seg, (_SEG, r))
    sid = lax.broadcasted_iota(jnp.int32, (_SEG, r), 0)
    ohb = (seg_b == sid).astype(jnp.bfloat16)

    @pl.when(phase == 0)
    def _accumulate_sums():
        x = x_ref[...]
        xb = x.astype(jnp.bfloat16)
        ones_db = jnp.ones((x.shape[1], 1), jnp.bfloat16)
        ssq = _dot(xb * xb, ones_db, ((1,), (0,)))
        scale = 1.0 / (jnp.sqrt(ssq) + 1e-8)
        xnb = xb * scale.astype(jnp.bfloat16)
        xn_ref[pl.ds(i * r, r), :] = xnb
        sums_ref[...] += _dot(ohb, xnb, ((1,), (0,)))

    @pl.when((phase == 1) & (i == 0))
    def _finalize():
        ones2 = jnp.ones((2, 1), jnp.float32)
        cnt = _dot(cnt32_ref[...], ones2, ((0,), (0,)))[0:_SEG, :]
        cnt_safe = jnp.maximum(cnt, 1.0)
        mu = sums_ref[...] / cnt_safe
        sums_ref[...] = mu
        present = (cnt > 0.0).astype(jnp.float32)

        sbid = lax.broadcasted_iota(jnp.int32, (_NS, _SEG), 0)
        segid2 = lax.broadcasted_iota(jnp.int32, (_NS, _SEG), 1)
        sb_oh = (segid2 // _NL == sbid).astype(jnp.float32)
        m_sb = _dot(sb_oh, present, ((1,), (0,)))
        m_safe = jnp.maximum(m_sb, 1.0)
        m_per_seg = _dot(sb_oh, m_safe, ((0,), (0,)))
        coef_ref[...] = present / (m_per_seg * cnt_safe)

        pts_sb = _dot(sb_oh, cnt, ((1,), (0,)))
        bval = jnp.sum((pts_sb > 0.0).astype(jnp.float32))

        segiota = lax.broadcasted_iota(jnp.int32, (_SEG, 1), 0).astype(jnp.float32)
        mu_push = mu + (1.0 - present) * (1.0e6 + 1.0e4 * segiota)
        push_total = jnp.float32(0.0)
        eye = (lax.broadcasted_iota(jnp.int32, (_NL, _NL), 0)
               == lax.broadcasted_iota(jnp.int32, (_NL, _NL), 1))
        for s in range(_NS):
            mus = mu_push[s * _NL:(s + 1) * _NL, :]
            p_col = present[s * _NL:(s + 1) * _NL, :]
            pd = jnp.sum(jnp.abs(mus[:, None, :] - mus[None, :, :]), axis=2)
            dists = jnp.maximum(2.0 * _DELTA_D - pd, 0.0)
            dm = jnp.where(eye, 0.0, dists)
            ms = jnp.sum(p_col)
            denom = jnp.where(ms > 1.0, ms * (ms - 1.0), 1.0)
            push_total = push_total + jnp.sum(dm * dm) / denom
        meta_ref[0] = push_total
        meta_ref[1] = bval

    @pl.when(phase == 1)
    def _accumulate_pull():
        xnb = xn_ref[pl.ds(i * r, r), :]
        mub = sums_ref[...].astype(jnp.bfloat16)
        musel = _dot(ohb, mub, ((0,), (0,)))
        diff = jnp.abs(musel.astype(jnp.bfloat16) - xnb)
        ones_db = jnp.ones((diff.shape[1], 1), jnp.bfloat16)
        d = _dot(diff, ones_db, ((1,), (0,)))
        t = jnp.maximum(d - _DELTA_V, 0.0)
        term = (t * t).astype(jnp.bfloat16)
        pull_ref[...] += _dot(ohb, term, ((1,), (0,)))

    @pl.when((phase == 1) & (i == nb - 1))
    def _final():
        lp = jnp.sum(pull_ref[...] * coef_ref[...])
        loss = (lp + meta_ref[0]) / meta_ref[1]
        out_ref[...] = jnp.broadcast_to(loss, (1, 1))


def kernel(outputs, labels, subbatch_indices):
    n, d = outputs.shape
    r = 4000
    nb = n // r
    assert n % r == 0

    grain = _SC_WORKERS * 128
    npad = (n + grain - 1) // grain * grain
    pad = npad - n
    lab_pad = jnp.concatenate([labels, jnp.zeros((pad,), jnp.int32)])
    sb_pad = jnp.concatenate([subbatch_indices, jnp.full((pad,), _NS, jnp.int32)])
    cnt32 = _sc_histogram(lab_pad, sb_pad, npad)

    lab3 = labels.reshape(nb, 1, r)
    sb3 = subbatch_indices.reshape(nb, 1, r)

    body = functools.partial(_loss_body, nb=nb, r=r)
    out = pl.pallas_call(
        body,
        grid=(2, nb),
        in_specs=[
            pl.BlockSpec((1, 1, r), lambda p, i: (i, 0, 0)),
            pl.BlockSpec((1, 1, r), lambda p, i: (i, 0, 0)),
            pl.BlockSpec((r, d), lambda p, i: (i * (1 - p), 0)),
            pl.BlockSpec((2, _HBINS), lambda p, i: (0, 0)),
        ],
        out_specs=pl.BlockSpec((1, 1), lambda p, i: (0, 0)),
        out_shape=jax.ShapeDtypeStruct((1, 1), jnp.float32),
        scratch_shapes=[
            pltpu.VMEM((_SEG, d), jnp.float32),
            pltpu.VMEM((_SEG, 1), jnp.float32),
            pltpu.VMEM((_SEG, 1), jnp.float32),
            pltpu.SMEM((2,), jnp.float32),
            pltpu.VMEM((n, d), jnp.bfloat16),
        ],
    )(lab3, sb3, outputs, cnt32)
    return out[0, 0]


# speedup vs baseline: 1.0006x; 1.0006x over previous
import functools

import jax
import jax.numpy as jnp
from jax import lax
from jax.experimental import pallas as pl
from jax.experimental.pallas import tpu as pltpu
from jax.experimental.pallas import tpu_sc as plsc

_DELTA_V = 0.5
_DELTA_D = 1.5
_NL = 20
_NS = 8
_SEG = _NL * _NS

_SC_WORKERS = 32
_LANES = 16
_HBINS = 192


def _sc_histogram(lab_pad, sb_pad, npad):
    chunk = npad // _SC_WORKERS
    rows = chunk // 128
    mesh = plsc.VectorSubcoreMesh(core_axis_name="c", subcore_axis_name="s")

    @functools.partial(
        pl.kernel, mesh=mesh,
        out_type=jax.ShapeDtypeStruct((2, _HBINS), jnp.float32),
        scratch_types=[
            pltpu.VMEM((chunk,), jnp.int32),
            pltpu.VMEM((chunk,), jnp.int32),
            pltpu.VMEM((rows, 128), jnp.int32),
            pltpu.VMEM((128,), jnp.float32),
            pltpu.VMEM((_HBINS,), jnp.float32),
            pltpu.VMEM_SHARED((_HBINS,), jnp.float32),
            pltpu.SemaphoreType.DMA,
        ],
    )
    def hist(lab_hbm, sb_hbm, out_hbm,
             lab_v, sb_v, segs_v, ones_v, bins_v, shared_v, sem):
        cid = lax.axis_index("c")
        sid = lax.axis_index("s")
        wid = sid * 2 + cid
        base = wid * chunk
        pltpu.sync_copy(lab_hbm.at[pl.ds(base, chunk)], lab_v)
        pltpu.sync_copy(sb_hbm.at[pl.ds(base, chunk)], sb_v)

        zeros16 = jnp.zeros((_LANES,), jnp.float32)
        ones16 = jnp.ones((_LANES,), jnp.float32)
        for t in range(128 // _LANES):
            ones_v[pl.ds(t * _LANES, _LANES)] = ones16
        for t in range(_HBINS // _LANES):
            bins_v[pl.ds(t * _LANES, _LANES)] = zeros16

        for t in range(chunk // _LANES):
            lab16 = lab_v[pl.ds(t * _LANES, _LANES)]
            sb16 = sb_v[pl.ds(t * _LANES, _LANES)]
            segs_v[t // 8, pl.ds((t % 8) * _LANES, _LANES)] = sb16 * _NL + lab16

        @pl.when(sid == 0)
        def _zero_shared():
            pltpu.sync_copy(bins_v, shared_v)
        plsc.subcore_barrier()

        copies = [pltpu.async_copy(ones_v, shared_v.at[segs_v.at[j]], sem,
                                   add=True)
                  for j in range(rows)]
        for c in copies:
            c.wait()
        plsc.subcore_barrier()

        @pl.when(sid == 0)
        def _write_out():
            pltpu.sync_copy(shared_v, bins_v)
            pltpu.sync_copy(bins_v, out_hbm.at[cid])

    return hist(lab_pad, sb_pad)


def _dot(a, b, dims):
    return lax.dot_general(a, b, (dims, ((), ())),
                           preferred_element_type=jnp.float32)


def _loss_body(lab_ref, sb_ref, x_ref, cnt32_ref, out_ref,
               sums_ref, pull_ref, coef_ref, meta_ref, xn_ref,
               *, nb, r):
    phase = pl.program_id(0)
    i = pl.program_id(1)

    @pl.when((phase == 0) & (i == 0))
    def _init():
        sums_ref[...] = jnp.zeros_like(sums_ref)
        pull_ref[...] = jnp.zeros_like(pull_ref)

    lab = lab_ref[0]
    sb = sb_ref[0]
    seg = sb * _NL + lab
    seg_b = jnp.broadcast_to(seg, (_SEG, r))
    sid = lax.broadcasted_iota(jnp.int32, (_SEG, r), 0)
    ohb = (seg_b == sid).astype(jnp.bfloat16)

    @pl.when(phase == 0)
    def _accumulate_sums():
        x = x_ref[...]
        xb = x.astype(jnp.bfloat16)
        ones_db = jnp.ones((x.shape[1], 1), jnp.bfloat16)
        ssq = _dot(xb * xb, ones_db, ((1,), (0,)))
        scale = 1.0 / (jnp.sqrt(ssq) + 1e-8)
        xnb = xb * scale.astype(jnp.bfloat16)
        xn_ref[pl.ds(i * r, r), :] = xnb
        sums_ref[...] += _dot(ohb, xnb, ((1,), (0,)))

    @pl.when((phase == 1) & (i == 0))
    def _finalize():
        ones2 = jnp.ones((2, 1), jnp.float32)
        cnt = _dot(cnt32_ref[...], ones2, ((0,), (0,)))[0:_SEG, :]
        cnt_safe = jnp.maximum(cnt, 1.0)
        mu = sums_ref[...] / cnt_safe
        sums_ref[...] = mu
        present = (cnt > 0.0).astype(jnp.float32)

        sbid = lax.broadcasted_iota(jnp.int32, (_NS, _SEG), 0)
        segid2 = lax.broadcasted_iota(jnp.int32, (_NS, _SEG), 1)
        sb_oh = (segid2 // _NL == sbid).astype(jnp.float32)
        m_sb = _dot(sb_oh, present, ((1,), (0,)))
        m_safe = jnp.maximum(m_sb, 1.0)
        m_per_seg = _dot(sb_oh, m_safe, ((0,), (0,)))
        coef_ref[...] = present / (m_per_seg * cnt_safe)

        pts_sb = _dot(sb_oh, cnt, ((1,), (0,)))
        bval = jnp.sum((pts_sb > 0.0).astype(jnp.float32))

        segiota = lax.broadcasted_iota(jnp.int32, (_SEG, 1), 0).astype(jnp.float32)
        mu_push = mu + (1.0 - present) * (1.0e6 + 1.0e4 * segiota)
        push_total = jnp.float32(0.0)
        eye = (lax.broadcasted_iota(jnp.int32, (_NL, _NL), 0)
               == lax.broadcasted_iota(jnp.int32, (_NL, _NL), 1))
        for s in range(_NS):
            mus = mu_push[s * _NL:(s + 1) * _NL, :]
            p_col = present[s * _NL:(s + 1) * _NL, :]
            pd = jnp.sum(jnp.abs(mus[:, None, :] - mus[None, :, :]), axis=2)
            dists = jnp.maximum(2.0 * _DELTA_D - pd, 0.0)
            dm = jnp.where(eye, 0.0, dists)
            ms = jnp.sum(p_col)
            denom = jnp.where(ms > 1.0, ms * (ms - 1.0), 1.0)
            push_total = push_total + jnp.sum(dm * dm) / denom
        meta_ref[0] = push_total
        meta_ref[1] = bval

    @pl.when(phase == 1)
    def _accumulate_pull():
        xnb = xn_ref[pl.ds(i * r, r), :]
        mub = sums_ref[...].astype(jnp.bfloat16)
        musel = _dot(ohb, mub, ((0,), (0,)))
        diff = jnp.abs(musel.astype(jnp.bfloat16) - xnb)
        ones_db = jnp.ones((diff.shape[1], 1), jnp.bfloat16)
        d = _dot(diff, ones_db, ((1,), (0,)))
        t = jnp.maximum(d - _DELTA_V, 0.0)
        term = (t * t).astype(jnp.bfloat16)
        pull_ref[...] += _dot(ohb, term, ((1,), (0,)))

    @pl.when((phase == 1) & (i == nb - 1))
    def _final():
        lp = jnp.sum(pull_ref[...] * coef_ref[...])
        loss = (lp + meta_ref[0]) / meta_ref[1]
        out_ref[...] = jnp.broadcast_to(loss, (1, 1))


def kernel(outputs, labels, subbatch_indices):
    n, d = outputs.shape
    r = 4000
    nb = n // r
    assert n % r == 0

    grain = _SC_WORKERS * 128
    npad = (n + grain - 1) // grain * grain
    pad = npad - n
    lab_pad = jnp.concatenate([labels, jnp.zeros((pad,), jnp.int32)])
    sb_pad = jnp.concatenate([subbatch_indices, jnp.full((pad,), _NS, jnp.int32)])
    cnt32 = _sc_histogram(lab_pad, sb_pad, npad)

    lab3 = labels.reshape(nb, 1, r)
    sb3 = subbatch_indices.reshape(nb, 1, r)

    body = functools.partial(_loss_body, nb=nb, r=r)
    out = pl.pallas_call(
        body,
        grid=(2, nb),
        in_specs=[
            pl.BlockSpec((1, 1, r), lambda p, i: (i, 0, 0)),
            pl.BlockSpec((1, 1, r), lambda p, i: (i, 0, 0)),
            pl.BlockSpec((r, d), lambda p, i: (i * (1 - p), 0)),
            pl.BlockSpec((2, _HBINS), lambda p, i: (0, 0)),
        ],
        out_specs=pl.BlockSpec((1, 1), lambda p, i: (0, 0)),
        out_shape=jax.ShapeDtypeStruct((1, 1), jnp.float32),
        scratch_shapes=[
            pltpu.VMEM((_SEG, d), jnp.float32),
            pltpu.VMEM((_SEG, 1), jnp.float32),
            pltpu.VMEM((_SEG, 1), jnp.float32),
            pltpu.SMEM((2,), jnp.float32),
            pltpu.VMEM((n, d), jnp.bfloat16),
        ],
    )(lab3, sb3, outputs, cnt32)
    return out[0, 0]


# speedup vs baseline: 1.1262x; 1.1255x over previous
import functools

import jax
import jax.numpy as jnp
from jax import lax
from jax.experimental import pallas as pl
from jax.experimental.pallas import tpu as pltpu
from jax.experimental.pallas import tpu_sc as plsc

_DELTA_V = 0.5
_DELTA_D = 1.5
_NL = 20
_NS = 8
_SEG = _NL * _NS

_SC_WORKERS = 32
_LANES = 16
_HBINS = 192


def _sc_histogram(lab_pad, sb_pad, npad):
    chunk = npad // _SC_WORKERS
    rows = chunk // 128
    mesh = plsc.VectorSubcoreMesh(core_axis_name="c", subcore_axis_name="s")

    @functools.partial(
        pl.kernel, mesh=mesh,
        out_type=jax.ShapeDtypeStruct((2, _HBINS), jnp.float32),
        scratch_types=[
            pltpu.VMEM((chunk,), jnp.int32),
            pltpu.VMEM((chunk,), jnp.int32),
            pltpu.VMEM((rows, 128), jnp.int32),
            pltpu.VMEM((128,), jnp.float32),
            pltpu.VMEM((_HBINS,), jnp.float32),
            pltpu.VMEM_SHARED((_HBINS,), jnp.float32),
            pltpu.SemaphoreType.DMA,
        ],
    )
    def hist(lab_hbm, sb_hbm, out_hbm,
             lab_v, sb_v, segs_v, ones_v, bins_v, shared_v, sem):
        cid = lax.axis_index("c")
        sid = lax.axis_index("s")
        wid = sid * 2 + cid
        base = wid * chunk
        pltpu.sync_copy(lab_hbm.at[pl.ds(base, chunk)], lab_v)
        pltpu.sync_copy(sb_hbm.at[pl.ds(base, chunk)], sb_v)

        zeros16 = jnp.zeros((_LANES,), jnp.float32)
        ones16 = jnp.ones((_LANES,), jnp.float32)
        for t in range(128 // _LANES):
            ones_v[pl.ds(t * _LANES, _LANES)] = ones16
        for t in range(_HBINS // _LANES):
            bins_v[pl.ds(t * _LANES, _LANES)] = zeros16

        for t in range(chunk // _LANES):
            lab16 = lab_v[pl.ds(t * _LANES, _LANES)]
            sb16 = sb_v[pl.ds(t * _LANES, _LANES)]
            segs_v[t // 8, pl.ds((t % 8) * _LANES, _LANES)] = sb16 * _NL + lab16

        @pl.when(sid == 0)
        def _zero_shared():
            pltpu.sync_copy(bins_v, shared_v)
        plsc.subcore_barrier()

        copies = [pltpu.async_copy(ones_v, shared_v.at[segs_v.at[j]], sem,
                                   add=True)
                  for j in range(rows)]
        for c in copies:
            c.wait()
        plsc.subcore_barrier()

        @pl.when(sid == 0)
        def _write_out():
            pltpu.sync_copy(shared_v, bins_v)
            pltpu.sync_copy(bins_v, out_hbm.at[cid])

    return hist(lab_pad, sb_pad)


def _dot(a, b, dims):
    return lax.dot_general(a, b, (dims, ((), ())),
                           preferred_element_type=jnp.float32)


def _loss_body(lab_ref, sb_ref, x_ref, cnt32_ref, out_ref,
               sums_ref, pull_ref, coef_ref, meta_ref, xn_ref,
               *, nb, r):
    phase = pl.program_id(0)
    i = pl.program_id(1)

    @pl.when((phase == 0) & (i == 0))
    def _init():
        sums_ref[...] = jnp.zeros_like(sums_ref)
        pull_ref[...] = jnp.zeros_like(pull_ref)

    lab = lab_ref[0]
    sb = sb_ref[0]
    seg = sb * _NL + lab
    seg_b = jnp.broadcast_to(seg, (_SEG, r))
    sid = lax.broadcasted_iota(jnp.int32, (_SEG, r), 0)
    ohb = (seg_b == sid).astype(jnp.bfloat16)

    @pl.when(phase == 0)
    def _accumulate_sums():
        x = x_ref[...]
        xb = x.astype(jnp.bfloat16)
        ones_db = jnp.ones((x.shape[1], 1), jnp.bfloat16)
        ssq = _dot(xb * xb, ones_db, ((1,), (0,)))
        scale = 1.0 / (jnp.sqrt(ssq) + 1e-8)
        xnb = xb * scale.astype(jnp.bfloat16)
        xn_ref[pl.ds(i * r, r), :] = xnb
        sums_ref[...] += _dot(ohb, xnb, ((1,), (0,)))

    @pl.when((phase == 1) & (i == 0))
    def _finalize():
        ones2 = jnp.ones((2, 1), jnp.float32)
        cnt = _dot(cnt32_ref[...], ones2, ((0,), (0,)))[0:_SEG, :]
        cnt_safe = jnp.maximum(cnt, 1.0)
        mu = sums_ref[...] / cnt_safe
        sums_ref[...] = mu
        present = (cnt > 0.0).astype(jnp.float32)

        sbid = lax.broadcasted_iota(jnp.int32, (_NS, _SEG), 0)
        segid2 = lax.broadcasted_iota(jnp.int32, (_NS, _SEG), 1)
        sb_oh = (segid2 // _NL == sbid).astype(jnp.float32)
        m_sb = _dot(sb_oh, present, ((1,), (0,)))
        m_safe = jnp.maximum(m_sb, 1.0)
        m_per_seg = _dot(sb_oh, m_safe, ((0,), (0,)))
        coef_ref[...] = present / (m_per_seg * cnt_safe)

        pts_sb = _dot(sb_oh, cnt, ((1,), (0,)))
        bval = jnp.sum((pts_sb > 0.0).astype(jnp.float32))

        segiota = lax.broadcasted_iota(jnp.int32, (_SEG, 1), 0).astype(jnp.float32)
        mu_push = mu + (1.0 - present) * (1.0e6 + 1.0e4 * segiota)
        push_total = jnp.float32(0.0)
        eye = (lax.broadcasted_iota(jnp.int32, (_NL, _NL), 0)
               == lax.broadcasted_iota(jnp.int32, (_NL, _NL), 1))
        for s in range(_NS):
            mus = mu_push[s * _NL:(s + 1) * _NL, :]
            p_col = present[s * _NL:(s + 1) * _NL, :]
            pd = jnp.sum(jnp.abs(mus[:, None, :] - mus[None, :, :]), axis=2)
            dists = jnp.maximum(2.0 * _DELTA_D - pd, 0.0)
            dm = jnp.where(eye, 0.0, dists)
            ms = jnp.sum(p_col)
            denom = jnp.where(ms > 1.0, ms * (ms - 1.0), 1.0)
            push_total = push_total + jnp.sum(dm * dm) / denom
        meta_ref[0] = push_total
        meta_ref[1] = bval

    @pl.when(phase == 1)
    def _accumulate_pull():
        xnb = xn_ref[pl.ds(i * r, r), :]
        mub = sums_ref[...].astype(jnp.bfloat16)
        musel = _dot(ohb, mub, ((0,), (0,)))
        diff = jnp.abs(musel.astype(jnp.bfloat16) - xnb)
        ones_db = jnp.ones((diff.shape[1], 1), jnp.bfloat16)
        d = _dot(diff, ones_db, ((1,), (0,)))
        t = jnp.maximum(d - _DELTA_V, 0.0)
        term = (t * t).astype(jnp.bfloat16)
        pull_ref[...] += _dot(ohb, term, ((1,), (0,)))

    @pl.when((phase == 1) & (i == nb - 1))
    def _final():
        lp = jnp.sum(pull_ref[...] * coef_ref[...])
        loss = (lp + meta_ref[0]) / meta_ref[1]
        out_ref[...] = jnp.broadcast_to(loss, (1, 1))


def kernel(outputs, labels, subbatch_indices):
    n, d = outputs.shape
    r = 10000
    nb = n // r
    assert n % r == 0

    grain = _SC_WORKERS * 128
    npad = (n + grain - 1) // grain * grain
    pad = npad - n
    lab_pad = jnp.concatenate([labels, jnp.zeros((pad,), jnp.int32)])
    sb_pad = jnp.concatenate([subbatch_indices, jnp.full((pad,), _NS, jnp.int32)])
    cnt32 = _sc_histogram(lab_pad, sb_pad, npad)

    lab3 = labels.reshape(nb, 1, r)
    sb3 = subbatch_indices.reshape(nb, 1, r)

    body = functools.partial(_loss_body, nb=nb, r=r)
    out = pl.pallas_call(
        body,
        grid=(2, nb),
        in_specs=[
            pl.BlockSpec((1, 1, r), lambda p, i: (i, 0, 0)),
            pl.BlockSpec((1, 1, r), lambda p, i: (i, 0, 0)),
            pl.BlockSpec((r, d), lambda p, i: (i * (1 - p), 0)),
            pl.BlockSpec((2, _HBINS), lambda p, i: (0, 0)),
        ],
        out_specs=pl.BlockSpec((1, 1), lambda p, i: (0, 0)),
        out_shape=jax.ShapeDtypeStruct((1, 1), jnp.float32),
        scratch_shapes=[
            pltpu.VMEM((_SEG, d), jnp.float32),
            pltpu.VMEM((_SEG, 1), jnp.float32),
            pltpu.VMEM((_SEG, 1), jnp.float32),
            pltpu.SMEM((2,), jnp.float32),
            pltpu.VMEM((n, d), jnp.bfloat16),
        ],
    )(lab3, sb3, outputs, cnt32)
    return out[0, 0]
